# baseline (speedup 1.0000x reference)
import jax
import jax.numpy as jnp
from jax import lax
from jax.experimental import pallas as pl
from jax.experimental.pallas import tpu as pltpu
from jax.experimental.pallas import tpu_sc as plsc

N_ENT = 1000000
N_FACTORS = 16
BATCH = 16384

ROWS_PER_LINE = 8
N_LINES = N_ENT // ROWS_PER_LINE
NUM_CORES = 2
NUM_SUBCORES = 16
NUM_WORKERS = NUM_CORES * NUM_SUBCORES
B_PER_W = BATCH // NUM_WORKERS
CHUNK = 128
N_CHUNKS = B_PER_W // CHUNK
GROUPS_PER_CHUNK = CHUNK // 16


def _sc_body(heads_hbm, tails_hbm, emb_h_hbm, emb_t_hbm, out_hbm,
             idx_h, idx_t, lin_h, lin_t,
             bufs_h, bufs_t, out_v, sems):
    wid = lax.axis_index("s") * NUM_CORES + lax.axis_index("c")

    pltpu.sync_copy(heads_hbm.at[pl.ds(wid * N_CHUNKS, N_CHUNKS)], idx_h)
    pltpu.sync_copy(tails_hbm.at[pl.ds(wid * N_CHUNKS, N_CHUNKS)], idx_t)

    for c in range(N_CHUNKS):
        for v in range(GROUPS_PER_CHUNK):
            s = pl.ds(v * 16, 16)
            lin_h[c, s] = idx_h[c, s] >> 3
            lin_t[c, s] = idx_t[c, s] >> 3

    def fire(c):
        p = c % 2
        return (
            pltpu.async_copy(emb_h_hbm.at[lin_h.at[c]], bufs_h[p], sems[p]),
            pltpu.async_copy(emb_t_hbm.at[lin_t.at[c]], bufs_t[p], sems[p]),
        )

    lane = lax.iota(jnp.int32, 16)
    inflight = fire(0)
    for c in range(N_CHUNKS):
        if c + 1 < N_CHUNKS:
            nxt = fire(c + 1)
        for cp in inflight:
            cp.wait()
        bh, bt = bufs_h[c % 2], bufs_t[c % 2]
        for g in range(GROUPS_PER_CHUNK):
            j0 = g * 16
            hidx = idx_h[c, pl.ds(j0, 16)]
            tidx = idx_t[c, pl.ds(j0, 16)]
            hcol = (hidx & 7) * 16
            tcol = (tidx & 7) * 16
            rows = j0 + lane
            acc = jnp.zeros((16,), jnp.float32)
            for d in range(N_FACTORS):
                hv = plsc.load_gather(bh, [rows, hcol + d])
                tv = plsc.load_gather(bt, [rows, tcol + d])
                acc = acc + hv * tv
            out_v[pl.ds(c * CHUNK + j0, 16)] = acc
        if c + 1 < N_CHUNKS:
            inflight = nxt

    pltpu.sync_copy(out_v, out_hbm.at[pl.ds(wid * B_PER_W, B_PER_W)])


@jax.jit
def _run(heads2d, tails2d, emb_h_lines, emb_t_lines):
    mesh = plsc.VectorSubcoreMesh(core_axis_name="c", subcore_axis_name="s")
    f = pl.kernel(
        _sc_body,
        mesh=mesh,
        compiler_params=pltpu.CompilerParams(needs_layout_passes=False),
        out_type=jax.ShapeDtypeStruct((BATCH,), jnp.float32),
        scratch_types=[
            pltpu.VMEM((N_CHUNKS, CHUNK), jnp.int32),
            pltpu.VMEM((N_CHUNKS, CHUNK), jnp.int32),
            pltpu.VMEM((N_CHUNKS, CHUNK), jnp.int32),
            pltpu.VMEM((N_CHUNKS, CHUNK), jnp.int32),
            [pltpu.VMEM((CHUNK, 128), jnp.float32) for _ in range(2)],
            [pltpu.VMEM((CHUNK, 128), jnp.float32) for _ in range(2)],
            pltpu.VMEM((B_PER_W,), jnp.float32),
            [pltpu.SemaphoreType.DMA for _ in range(2)],
        ],
    )
    return f(heads2d, tails2d, emb_h_lines, emb_t_lines)


def kernel(heads, tails, emb_heads, emb_tails):
    heads2d = heads.astype(jnp.int32).reshape(NUM_WORKERS * N_CHUNKS, CHUNK)
    tails2d = tails.astype(jnp.int32).reshape(NUM_WORKERS * N_CHUNKS, CHUNK)
    emb_h_lines = emb_heads.reshape(N_LINES, ROWS_PER_LINE * N_FACTORS)
    emb_t_lines = emb_tails.reshape(N_LINES, ROWS_PER_LINE * N_FACTORS)
    return _run(heads2d, tails2d, emb_h_lines, emb_t_lines)


# speedup vs baseline: 5.9443x; 5.9443x over previous
import jax
import jax.numpy as jnp
from jax import lax
from jax.experimental import pallas as pl
from jax.experimental.pallas import tpu as pltpu
from jax.experimental.pallas import tpu_sc as plsc

N_ENT = 1000000
N_FACTORS = 16
BATCH = 16384

NUM_CORES = 2
NUM_SUBCORES = 16
NUM_WORKERS = NUM_CORES * NUM_SUBCORES
B_PER_W = BATCH // NUM_WORKERS
NSTAGES = B_PER_W // 16
LAST_TILE = 7811
TAIL0 = 999936


def _sc_body(heads_hbm, tails_hbm, th_hbm, tt_hbm, tailh_hbm, tailt_hbm,
             out_hbm, idx_h, idx_t, blk_h, blk_t, tail_h, tail_t, out_v,
             sem0, sem1):
    wid = lax.axis_index("s") * NUM_CORES + lax.axis_index("c")

    pltpu.sync_copy(heads_hbm.at[wid], idx_h)
    pltpu.sync_copy(tails_hbm.at[wid], idx_t)
    pltpu.sync_copy(tailh_hbm, tail_h)
    pltpu.sync_copy(tailt_hbm, tail_t)

    lane = lax.iota(jnp.int32, 16)

    def issue(stage, slab, bank, sem):
        hv = idx_h[pl.ds(stage * 16, 16)]
        tv = idx_t[pl.ds(stage * 16, 16)]
        for l in range(16):
            et = jnp.minimum(hv[l] >> 7, LAST_TILE)
            ct = pl.multiple_of(et * 128, 128)
            pltpu.async_copy(th_hbm.at[slab, :, pl.ds(ct, 128)],
                             blk_h.at[bank * 16 + l], sem)
            eu = jnp.minimum(tv[l] >> 7, LAST_TILE)
            cu = pl.multiple_of(eu * 128, 128)
            pltpu.async_copy(tt_hbm.at[slab, :, pl.ds(cu, 128)],
                             blk_t.at[bank * 16 + l], sem)

    def drain(sem):
        dummy = th_hbm.at[0, :, pl.ds(0, 128)]
        for _ in range(32):
            pltpu.make_async_copy(dummy, blk_h.at[0], sem).wait()

    def compute(stage, slab, bank, first):
        hv = idx_h[pl.ds(stage * 16, 16)]
        tv = idx_t[pl.ds(stage * 16, 16)]
        htile = jnp.minimum(hv >> 7, LAST_TILE)
        ttile = jnp.minimum(tv >> 7, LAST_TILE)
        hcol = jnp.minimum(hv - htile * 128, 127)
        tcol = jnp.minimum(tv - ttile * 128, 127)
        hmask = hv >= TAIL0
        tmask = tv >= TAIL0
        hct = jnp.clip(hv - TAIL0, 0, 127)
        tct = jnp.clip(tv - TAIL0, 0, 127)
        slots = bank * 16 + lane
        acc = jnp.zeros((16,), jnp.float32)
        for d in range(8):
            dfull = jnp.full((16,), d, jnp.int32)
            sfull = jnp.full((16,), slab, jnp.int32)
            hb = plsc.load_gather(blk_h, [slots, dfull, hcol])
            tb = plsc.load_gather(blk_t, [slots, dfull, tcol])
            hx = plsc.load_gather(tail_h, [sfull, dfull, hct])
            tx = plsc.load_gather(tail_t, [sfull, dfull, tct])
            he = jnp.where(hmask, hx, hb)
            te = jnp.where(tmask, tx, tb)
            acc = acc + he * te
        s = pl.ds(stage * 16, 16)
        if first:
            out_v[s] = acc
        else:
            out_v[s] = out_v[s] + acc

    issue(0, 0, 0, sem0)

    def body(i, carry):
        issue(i, 1, 1, sem1)
        drain(sem0)
        compute(i, 0, 0, True)

        @pl.when(i < NSTAGES - 1)
        def _():
            issue(i + 1, 0, 0, sem0)
        drain(sem1)
        compute(i, 1, 1, False)
        return carry

    lax.fori_loop(0, NSTAGES, body, 0)

    pltpu.sync_copy(out_v, out_hbm.at[pl.ds(wid * B_PER_W, B_PER_W)])


@jax.jit
def _run(heads2d, tails2d, th3, tt3, tailh, tailt):
    mesh = plsc.VectorSubcoreMesh(core_axis_name="c", subcore_axis_name="s")
    f = pl.kernel(
        _sc_body,
        mesh=mesh,
        compiler_params=pltpu.CompilerParams(needs_layout_passes=False),
        out_type=jax.ShapeDtypeStruct((BATCH,), jnp.float32),
        scratch_types=[
            pltpu.VMEM((B_PER_W,), jnp.int32),
            pltpu.VMEM((B_PER_W,), jnp.int32),
            pltpu.VMEM((32, 8, 128), jnp.float32),
            pltpu.VMEM((32, 8, 128), jnp.float32),
            pltpu.VMEM((2, 8, 128), jnp.float32),
            pltpu.VMEM((2, 8, 128), jnp.float32),
            pltpu.VMEM((B_PER_W,), jnp.float32),
            pltpu.SemaphoreType.DMA,
            pltpu.SemaphoreType.DMA,
        ],
    )
    return f(heads2d, tails2d, th3, tt3, tailh, tailt)


def kernel(heads, tails, emb_heads, emb_tails):
    heads2d = heads.astype(jnp.int32).reshape(NUM_WORKERS, B_PER_W)
    tails2d = tails.astype(jnp.int32).reshape(NUM_WORKERS, B_PER_W)
    th3 = jnp.swapaxes(emb_heads, 0, 1).reshape(2, 8, N_ENT)
    tt3 = jnp.swapaxes(emb_tails, 0, 1).reshape(2, 8, N_ENT)
    pad = ((0, 0), (0, 0), (0, 64))
    tailh = jnp.pad(th3[:, :, TAIL0:], pad)
    tailt = jnp.pad(tt3[:, :, TAIL0:], pad)
    return _run(heads2d, tails2d, th3, tt3, tailh, tailt)


# speedup vs baseline: 6.3178x; 1.0628x over previous
import jax
import jax.numpy as jnp
from jax import lax
from jax.experimental import pallas as pl
from jax.experimental.pallas import tpu as pltpu
from jax.experimental.pallas import tpu_sc as plsc

N_ENT = 1000000
N_FACTORS = 16
BATCH = 16384

NUM_CORES = 2
NUM_SUBCORES = 16
NUM_WORKERS = NUM_CORES * NUM_SUBCORES
B_PER_W = BATCH // NUM_WORKERS
NSTAGES = B_PER_W // 16
LAST_TILE = 7811
TAIL0 = 999936


def _sc_body(heads_hbm, tails_hbm, th_hbm, tt_hbm, tailh_hbm, tailt_hbm,
             out_hbm, idx_h, idx_t, blk_h, blk_t, tail_h, tail_t, out_v,
             sem0, sem1, sem2):
    wid = lax.axis_index("s") * NUM_CORES + lax.axis_index("c")

    pltpu.sync_copy(heads_hbm.at[wid], idx_h)
    pltpu.sync_copy(tails_hbm.at[wid], idx_t)
    pltpu.sync_copy(tailh_hbm, tail_h)
    pltpu.sync_copy(tailt_hbm, tail_t)

    lane = lax.iota(jnp.int32, 16)
    sems = (sem0, sem1, sem2)

    for i in range(NSTAGES):
        out_v[pl.ds(i * 16, 16)] = jnp.zeros((16,), jnp.float32)

    def issue(p, bank):
        stage = p >> 1
        slab = p & 1
        sem = sems[bank]
        hv = idx_h[pl.ds(stage * 16, 16)]
        tv = idx_t[pl.ds(stage * 16, 16)]
        for l in range(16):
            et = jnp.minimum(hv[l] >> 7, LAST_TILE)
            ct = pl.multiple_of(et * 128, 128)
            pltpu.async_copy(th_hbm.at[slab, :, pl.ds(ct, 128)],
                             blk_h.at[bank * 16 + l], sem)
            eu = jnp.minimum(tv[l] >> 7, LAST_TILE)
            cu = pl.multiple_of(eu * 128, 128)
            pltpu.async_copy(tt_hbm.at[slab, :, pl.ds(cu, 128)],
                             blk_t.at[bank * 16 + l], sem)

    def drain(bank):
        dummy = th_hbm.at[0, :, pl.ds(0, 128)]
        for _ in range(32):
            pltpu.make_async_copy(dummy, blk_h.at[0], sems[bank]).wait()

    def compute(p, bank):
        stage = p >> 1
        slab = p & 1
        hv = idx_h[pl.ds(stage * 16, 16)]
        tv = idx_t[pl.ds(stage * 16, 16)]
        htile = jnp.minimum(hv >> 7, LAST_TILE)
        ttile = jnp.minimum(tv >> 7, LAST_TILE)
        hcol = jnp.minimum(hv - htile * 128, 127)
        tcol = jnp.minimum(tv - ttile * 128, 127)
        hmask = hv >= TAIL0
        tmask = tv >= TAIL0
        hct = jnp.clip(hv - TAIL0, 0, 127)
        tct = jnp.clip(tv - TAIL0, 0, 127)
        slots = bank * 16 + lane
        sfull = jnp.full((16,), slab, jnp.int32)
        acc = jnp.zeros((16,), jnp.float32)
        for d in range(8):
            dfull = jnp.full((16,), d, jnp.int32)
            hb = plsc.load_gather(blk_h, [slots, dfull, hcol])
            tb = plsc.load_gather(blk_t, [slots, dfull, tcol])
            hx = plsc.load_gather(tail_h, [sfull, dfull, hct])
            tx = plsc.load_gather(tail_t, [sfull, dfull, tct])
            he = jnp.where(hmask, hx, hb)
            te = jnp.where(tmask, tx, tb)
            acc = acc + he * te
        s = pl.ds(stage * 16, 16)
        out_v[s] = out_v[s] + acc

    NPH = 2 * NSTAGES
    issue(0, 0)
    issue(1, 1)
    issue(2, 2)

    def body(j, carry):
        p = j * 3
        drain(0)
        compute(p, 0)
        issue(p + 3, 0)
        drain(1)
        compute(p + 1, 1)

        @pl.when(p + 4 < NPH)
        def _():
            issue(p + 4, 1)
        drain(2)
        compute(p + 2, 2)

        @pl.when(p + 5 < NPH)
        def _():
            issue(p + 5, 2)
        return carry

    lax.fori_loop(0, (NPH - 1) // 3, body, 0)
    drain(0)
    compute(NPH - 1, 0)

    pltpu.sync_copy(out_v, out_hbm.at[pl.ds(wid * B_PER_W, B_PER_W)])


@jax.jit
def _run(heads2d, tails2d, th3, tt3, tailh, tailt):
    mesh = plsc.VectorSubcoreMesh(core_axis_name="c", subcore_axis_name="s")
    f = pl.kernel(
        _sc_body,
        mesh=mesh,
        compiler_params=pltpu.CompilerParams(needs_layout_passes=False),
        out_type=jax.ShapeDtypeStruct((BATCH,), jnp.float32),
        scratch_types=[
            pltpu.VMEM((B_PER_W,), jnp.int32),
            pltpu.VMEM((B_PER_W,), jnp.int32),
            pltpu.VMEM((48, 8, 128), jnp.float32),
            pltpu.VMEM((48, 8, 128), jnp.float32),
            pltpu.VMEM((2, 8, 128), jnp.float32),
            pltpu.VMEM((2, 8, 128), jnp.float32),
            pltpu.VMEM((B_PER_W,), jnp.float32),
            pltpu.SemaphoreType.DMA,
            pltpu.SemaphoreType.DMA,
            pltpu.SemaphoreType.DMA,
        ],
    )
    return f(heads2d, tails2d, th3, tt3, tailh, tailt)


def kernel(heads, tails, emb_heads, emb_tails):
    heads2d = heads.astype(jnp.int32).reshape(NUM_WORKERS, B_PER_W)
    tails2d = tails.astype(jnp.int32).reshape(NUM_WORKERS, B_PER_W)
    th3 = jnp.swapaxes(emb_heads, 0, 1).reshape(2, 8, N_ENT)
    tt3 = jnp.swapaxes(emb_tails, 0, 1).reshape(2, 8, N_ENT)
    pad = ((0, 0), (0, 0), (0, 64))
    tailh = jnp.pad(th3[:, :, TAIL0:], pad)
    tailt = jnp.pad(tt3[:, :, TAIL0:], pad)
    return _run(heads2d, tails2d, th3, tt3, tailh, tailt)
